# baseline (speedup 1.0000x reference)
import functools

import jax
import jax.numpy as jnp
from jax import lax
from jax.experimental import pallas as pl
from jax.experimental.pallas import tpu as pltpu
from jax.experimental.pallas import tpu_sc as plsc

N = 4096
DIN = 384
DH = 512
TK = 6
BR = 256
NBB = N // BR
NSEG = 2
NQ = N // NSEG
NBQ = NQ // BR

_NC = 2
_NS = 16
_NW = _NC * _NS
_BQ = NQ * TK
_BPW = _BQ // _NW
_CH = 64
_NCHUNK = _BPW // _CH
_KBUF = 3


def _leaky(x):
    return jnp.where(x >= 0, x, 0.01 * x)


def _dot(a, b):
    return jnp.dot(a, b, preferred_element_type=jnp.float32)


def _topk_block(eh, et_full):
    scale = DH ** (-0.5)
    logits = lax.dot_general(eh * scale, et_full,
                             (((1,), (1,)), ((), ())),
                             preferred_element_type=jnp.float32)
    iota = lax.broadcasted_iota(jnp.int32, logits.shape, 1)
    vals, idxs = [], []
    for _ in range(TK):
        m = jnp.max(logits, axis=1, keepdims=True)
        jj = jnp.min(jnp.where(logits >= m, iota, N), axis=1, keepdims=True)
        vals.append(m)
        idxs.append(jj)
        logits = jnp.where(iota == jj, -jnp.inf, logits)
    v = jnp.concatenate(vals, axis=1)
    ji = jnp.concatenate(idxs, axis=1)
    e = jnp.exp(v - v[:, 0:1])
    return e / jnp.sum(e, axis=1, keepdims=True), ji


def _ka_body(data_ref, fc1w_ref, fc1b_ref, whw_ref, whb_ref, wtw_ref, wtb_ref,
             eh_ref, et_ref, prob_ref, idx_ref,
             h1_s, eh_s, et_s, csum_s):
    i = pl.program_id(0)

    @pl.when(i < NBB)
    def _phase0():
        h = _leaky(_dot(data_ref[...], fc1w_ref[...]) + fc1b_ref[...])
        h1_s[pl.ds(i * BR, BR), :] = h

        @pl.when(i == 0)
        def _():
            csum_s[...] = jnp.zeros_like(csum_s)

        csum_s[...] += jnp.sum(h, axis=0, keepdims=True)

    @pl.when(jnp.logical_and(i >= NBB, i < 2 * NBB))
    def _phase1():
        j = i - NBB
        x = (h1_s[pl.ds(j * BR, BR), :] + csum_s[...] * (1.0 / N)) * 0.5
        eh = _dot(x, whw_ref[...]) + whb_ref[...]
        et = _dot(x, wtw_ref[...]) + wtb_ref[...]
        eh_s[pl.ds(j * BR, BR), :] = eh
        et_s[pl.ds(j * BR, BR), :] = et
        eh_ref[...] = eh
        et_ref[...] = et

    @pl.when(i >= 2 * NBB)
    def _phase2():
        j = i - 2 * NBB
        p, ji = _topk_block(eh_s[pl.ds(j * BR, BR), :], et_s[...])
        prob_ref[...] = p
        idx_ref[...] = ji


def _kb_body(eh_ref, et_ref, prob_ref, idx_ref):
    p, ji = _topk_block(eh_ref[...], et_ref[...])
    prob_ref[...] = p
    idx_ref[...] = ji


def _sc_gather_body(table_hbm, idx_hbm, out_hbm, idx_v, *bufs_sems):
    bufs = bufs_sems[:_KBUF]
    gsem = bufs_sems[_KBUF:2 * _KBUF]
    ssem = bufs_sems[2 * _KBUF:3 * _KBUF]
    wid = lax.axis_index("s") * _NC + lax.axis_index("c")
    base = wid * _BPW
    pltpu.sync_copy(idx_hbm.at[pl.ds(base, _BPW)], idx_v)
    gh = [None] * _KBUF
    sh = [None] * _KBUF
    for b in range(_KBUF):
        gh[b] = pltpu.async_copy(
            table_hbm.at[idx_v.at[pl.ds(b * _CH, _CH)]], bufs[b], gsem[b])
    for c in range(_NCHUNK):
        b = c % _KBUF
        gh[b].wait()
        sh[b] = pltpu.async_copy(
            bufs[b], out_hbm.at[pl.ds(base + c * _CH, _CH)], ssem[b])
        if c + _KBUF < _NCHUNK:
            sh[b].wait()
            gh[b] = pltpu.async_copy(
                table_hbm.at[idx_v.at[pl.ds((c + _KBUF) * _CH, _CH)]],
                bufs[b], gsem[b])
    for c in range(max(0, _NCHUNK - _KBUF), _NCHUNK):
        sh[c % _KBUF].wait()


def _gather_rows(table, idx):
    mesh = plsc.VectorSubcoreMesh(
        core_axis_name="c", subcore_axis_name="s",
        num_cores=_NC, num_subcores=_NS)
    f = functools.partial(
        pl.kernel, mesh=mesh,
        out_type=jax.ShapeDtypeStruct((_BQ, DH), jnp.float32),
        scratch_types=[pltpu.VMEM((_BPW,), jnp.int32)]
        + [pltpu.VMEM((_CH, DH), jnp.float32) for _ in range(_KBUF)]
        + [pltpu.SemaphoreType.DMA for _ in range(2 * _KBUF)],
    )(_sc_gather_body)
    return f(table, idx)


def _kc_body(nb_ref, eh_ref, p_ref, l1w_ref, l1b_ref, l2w_ref, l2b_ref,
             a1w_ref, a1b_ref, a2w_ref, a2b_ref, emb_ref, g_ref):
    Nb = nb_ref[...]
    eh = eh_ref[...]
    p3 = p_ref[...].T[:, :, None]
    eh3 = eh[None, :, :]
    eh_r = p3 * Nb + (1.0 - p3) * eh3
    gate = jnp.tanh(eh3 + eh_r)
    ka = jnp.sum(Nb, axis=2) * jnp.sum(gate, axis=2)
    m = jnp.max(ka, axis=0, keepdims=True)
    e = jnp.exp(ka - m)
    kp = e / jnp.sum(e, axis=0, keepdims=True)
    eNh = jnp.sum(kp[:, :, None] * Nb, axis=0)
    s = _leaky(_dot(eh + eNh, l1w_ref[...]) + l1b_ref[...])
    bi = _leaky(_dot(eh * eNh, l2w_ref[...]) + l2b_ref[...])
    emb = s + bi
    emb_ref[...] = emb
    a1 = _leaky(_dot(emb, a1w_ref[...]) + a1b_ref[...])
    g_ref[...] = _dot(a1, a2w_ref[...]) + a2b_ref[...]


def _kr_body(*refs):
    es = refs[:NSEG]
    gs = refs[NSEG:2 * NSEG]
    (ng_ref, nbeta_ref, fcw_ref, fcb_ref, lg_ref, yp_ref, yh_ref) = refs[2 * NSEG:]
    m = jnp.max(gs[0][...])
    for g in gs[1:]:
        m = jnp.maximum(m, jnp.max(g[...]))
    denom = 0.0
    hr = jnp.zeros((1, DH), dtype=jnp.float32)
    for e, g in zip(es, gs):
        w = jnp.exp(g[...] - m)
        denom = denom + jnp.sum(w)
        hr = hr + jnp.sum(w * e[...], axis=0, keepdims=True)
    hr = hr / denom
    mu = jnp.mean(hr, axis=1, keepdims=True)
    var = jnp.mean((hr - mu) ** 2, axis=1, keepdims=True)
    hn = (hr - mu) / jnp.sqrt(var + 1e-5) * ng_ref[...] + nbeta_ref[...]
    lg = _dot(hn, fcw_ref[...]) + fcb_ref[...]
    lg_ref[...] = lg
    mm = jnp.max(lg, axis=1, keepdims=True)
    ee = jnp.exp(lg - mm)
    yp_ref[...] = ee / jnp.sum(ee, axis=1, keepdims=True)
    yh_ref[...] = jnp.where(lg[:, 1:2] > lg[:, 0:1], 1, 0).astype(jnp.int32)


def kernel(data, CT_data, fc1_W, fc1_b, Wh_W, Wh_b, Wt_W, Wt_b,
           lin1_W, lin1_b, lin2_W, lin2_b, att1_W, att1_b, att2_W, att2_b,
           norm_g, norm_beta, fc_W, fc_b):
    del CT_data
    x0 = jnp.squeeze(data, axis=0)
    r2 = lambda v: v.reshape(1, -1)
    full = lambda a, b: pl.BlockSpec((a, b), lambda i: (0, 0))

    ph1rows = lambda b: pl.BlockSpec(
        (BR, b), lambda i: (jnp.clip(i - NBB, 0, NBB - 1), 0))
    ph2rows = lambda b: pl.BlockSpec(
        (BR, b), lambda i: (jnp.clip(i - 2 * NBB, 0, NBQ - 1), 0))

    e_h, e_t, probs0, idx0 = pl.pallas_call(
        _ka_body,
        grid=(2 * NBB + NBQ,),
        in_specs=[pl.BlockSpec((BR, DIN), lambda i: (jnp.minimum(i, NBB - 1), 0)),
                  full(DIN, DH), full(1, DH),
                  full(DH, DH), full(1, DH),
                  full(DH, DH), full(1, DH)],
        out_specs=[ph1rows(DH), ph1rows(DH), ph2rows(TK), ph2rows(TK)],
        out_shape=[jax.ShapeDtypeStruct((N, DH), jnp.float32),
                   jax.ShapeDtypeStruct((N, DH), jnp.float32),
                   jax.ShapeDtypeStruct((NQ, TK), jnp.float32),
                   jax.ShapeDtypeStruct((NQ, TK), jnp.int32)],
        scratch_shapes=[pltpu.VMEM((N, DH), jnp.float32),
                        pltpu.VMEM((N, DH), jnp.float32),
                        pltpu.VMEM((N, DH), jnp.float32),
                        pltpu.VMEM((1, DH), jnp.float32)],
    )(x0, fc1_W, r2(fc1_b), Wh_W, r2(Wh_b), Wt_W, r2(Wt_b))

    def topk_seg(s):
        return pl.pallas_call(
            _kb_body,
            grid=(NBQ,),
            in_specs=[pl.BlockSpec((BR, DH), lambda i, s=s: (s * NBQ + i, 0)),
                      full(N, DH)],
            out_specs=[pl.BlockSpec((BR, TK), lambda i: (i, 0)),
                       pl.BlockSpec((BR, TK), lambda i: (i, 0))],
            out_shape=[jax.ShapeDtypeStruct((NQ, TK), jnp.float32),
                       jax.ShapeDtypeStruct((NQ, TK), jnp.int32)],
        )(e_h, e_t)

    probs = [probs0] + [None] * (NSEG - 1)
    idxs = [idx0] + [None] * (NSEG - 1)
    for s in range(1, NSEG):
        probs[s], idxs[s] = topk_seg(s)

    nbs = [_gather_rows(e_t, idxs[s].T.reshape(_BQ)).reshape(TK, NQ, DH)
           for s in range(NSEG)]

    wspecs = [full(DH, DH), full(1, DH), full(DH, DH), full(1, DH),
              full(DH, DH // 2), full(1, DH // 2), full(DH // 2, 1),
              full(1, 1)]
    wargs = (lin1_W, r2(lin1_b), lin2_W, r2(lin2_b),
             att1_W, r2(att1_b), att2_W, r2(att2_b))

    def combine_seg(s):
        return pl.pallas_call(
            _kc_body,
            grid=(NBQ,),
            in_specs=[pl.BlockSpec((TK, BR, DH), lambda i: (0, i, 0)),
                      pl.BlockSpec((BR, DH), lambda i, s=s: (s * NBQ + i, 0)),
                      pl.BlockSpec((BR, TK), lambda i: (i, 0))] + wspecs,
            out_specs=[pl.BlockSpec((BR, DH), lambda i: (i, 0)),
                       pl.BlockSpec((BR, 1), lambda i: (i, 0))],
            out_shape=[jax.ShapeDtypeStruct((NQ, DH), jnp.float32),
                       jax.ShapeDtypeStruct((NQ, 1), jnp.float32)],
        )(nbs[s], e_h, probs[s], *wargs)

    embs, gss = [], []
    for s in range(NSEG):
        emb_s, g_s = combine_seg(s)
        embs.append(emb_s)
        gss.append(g_s)

    full0 = lambda a, b: pl.BlockSpec((a, b), lambda: (0, 0))
    logits, y_prob, y_hat = pl.pallas_call(
        _kr_body,
        in_specs=[full0(NQ, DH)] * NSEG + [full0(NQ, 1)] * NSEG
                 + [full0(1, DH), full0(1, DH), full0(DH, 2), full0(1, 2)],
        out_specs=[full0(1, 2), full0(1, 2), full0(1, 1)],
        out_shape=[jax.ShapeDtypeStruct((1, 2), jnp.float32),
                   jax.ShapeDtypeStruct((1, 2), jnp.float32),
                   jax.ShapeDtypeStruct((1, 1), jnp.int32)],
    )(*embs, *gss, r2(norm_g), r2(norm_beta), fc_W, r2(fc_b))

    return (logits, y_prob, y_hat)


# speedup vs baseline: 1.0985x; 1.0985x over previous
import functools

import jax
import jax.numpy as jnp
from jax import lax
from jax.experimental import pallas as pl
from jax.experimental.pallas import tpu as pltpu
from jax.experimental.pallas import tpu_sc as plsc

N = 4096
DIN = 384
DH = 512
TK = 6
BR = 256
NBB = N // BR
NSEG = 2
NQ = N // NSEG
NBQ = NQ // BR

_NC = 2
_NS = 16
_NW = _NC * _NS
_BQ = NQ * TK
_BPW = _BQ // _NW
_CH = 64
_NCHUNK = _BPW // _CH
_KBUF = 3


def _leaky(x):
    return jnp.where(x >= 0, x, 0.01 * x)


def _dot(a, b):
    return jnp.dot(a, b, preferred_element_type=jnp.float32)


def _topk_block(eh, et_full):
    scale = DH ** (-0.5)
    logits = lax.dot_general(eh * scale, et_full,
                             (((1,), (1,)), ((), ())),
                             preferred_element_type=jnp.float32)
    iota = lax.broadcasted_iota(jnp.int32, logits.shape, 1)
    vals, idxs = [], []
    for _ in range(TK):
        m = jnp.max(logits, axis=1, keepdims=True)
        jj = jnp.min(jnp.where(logits >= m, iota, N), axis=1, keepdims=True)
        vals.append(m)
        idxs.append(jj)
        logits = jnp.where(iota == jj, -jnp.inf, logits)
    v = jnp.concatenate(vals, axis=1)
    ji = jnp.concatenate(idxs, axis=1)
    e = jnp.exp(v - v[:, 0:1])
    return e / jnp.sum(e, axis=1, keepdims=True), ji


def _ka_body(data_ref, fc1w_ref, fc1b_ref, whw_ref, whb_ref, wtw_ref, wtb_ref,
             eh_ref, et_ref, prob_ref, idx_ref,
             h1_s, eh_s, et_s, csum_s):
    i = pl.program_id(0)

    @pl.when(i < NBB)
    def _phase0():
        h = _leaky(_dot(data_ref[...], fc1w_ref[...]) + fc1b_ref[...])
        h1_s[pl.ds(i * BR, BR), :] = h

        @pl.when(i == 0)
        def _():
            csum_s[...] = jnp.zeros_like(csum_s)

        csum_s[...] += jnp.sum(h, axis=0, keepdims=True)

    @pl.when(jnp.logical_and(i >= NBB, i < 2 * NBB))
    def _phase1():
        j = i - NBB
        x = (h1_s[pl.ds(j * BR, BR), :] + csum_s[...] * (1.0 / N)) * 0.5
        eh = _dot(x, whw_ref[...]) + whb_ref[...]
        et = _dot(x, wtw_ref[...]) + wtb_ref[...]
        eh_s[pl.ds(j * BR, BR), :] = eh
        et_s[pl.ds(j * BR, BR), :] = et
        eh_ref[...] = eh
        et_ref[...] = et

    @pl.when(i >= 2 * NBB)
    def _phase2():
        j = i - 2 * NBB
        p, ji = _topk_block(eh_s[pl.ds(j * BR, BR), :], et_s[...])
        prob_ref[...] = p
        idx_ref[...] = ji


def _kb_body(eh_ref, et_ref, prob_ref, idx_ref):
    p, ji = _topk_block(eh_ref[...], et_ref[...])
    prob_ref[...] = p
    idx_ref[...] = ji


def _sc_gather_body(table_hbm, idx_hbm, out_hbm, idx_v, *bufs_sems):
    bufs = bufs_sems[:_KBUF]
    gsem = bufs_sems[_KBUF:2 * _KBUF]
    ssem = bufs_sems[2 * _KBUF:3 * _KBUF]
    wid = lax.axis_index("s") * _NC + lax.axis_index("c")
    base = wid * _BPW
    pltpu.sync_copy(idx_hbm.at[pl.ds(base, _BPW)], idx_v)
    gh = [None] * _KBUF
    sh = [None] * _KBUF
    for b in range(_KBUF):
        gh[b] = pltpu.async_copy(
            table_hbm.at[idx_v.at[pl.ds(b * _CH, _CH)]], bufs[b], gsem[b])
    for c in range(_NCHUNK):
        b = c % _KBUF
        gh[b].wait()
        sh[b] = pltpu.async_copy(
            bufs[b], out_hbm.at[pl.ds(base + c * _CH, _CH)], ssem[b])
        if c + _KBUF < _NCHUNK:
            sh[b].wait()
            gh[b] = pltpu.async_copy(
                table_hbm.at[idx_v.at[pl.ds((c + _KBUF) * _CH, _CH)]],
                bufs[b], gsem[b])
    for c in range(max(0, _NCHUNK - _KBUF), _NCHUNK):
        sh[c % _KBUF].wait()


def _gather_rows(table, idx):
    mesh = plsc.VectorSubcoreMesh(
        core_axis_name="c", subcore_axis_name="s",
        num_cores=_NC, num_subcores=_NS)
    f = functools.partial(
        pl.kernel, mesh=mesh,
        out_type=jax.ShapeDtypeStruct((_BQ, DH), jnp.float32),
        scratch_types=[pltpu.VMEM((_BPW,), jnp.int32)]
        + [pltpu.VMEM((_CH, DH), jnp.float32) for _ in range(_KBUF)]
        + [pltpu.SemaphoreType.DMA for _ in range(2 * _KBUF)],
    )(_sc_gather_body)
    return f(table, idx)


def _kc_body(nb_ref, eh_ref, p_ref, l1w_ref, l1b_ref, l2w_ref, l2b_ref,
             a1w_ref, a1b_ref, a2w_ref, a2b_ref, emb_ref, g_ref):
    emb, g = _combine_block(nb_ref, eh_ref, p_ref,
                            l1w_ref, l1b_ref, l2w_ref, l2b_ref,
                            a1w_ref, a1b_ref, a2w_ref, a2b_ref)
    emb_ref[...] = emb
    g_ref[...] = g


def _combine_block(nb_ref, eh_ref, p_ref, l1w_ref, l1b_ref, l2w_ref, l2b_ref,
                   a1w_ref, a1b_ref, a2w_ref, a2b_ref):
    Nb = nb_ref[...]
    eh = eh_ref[...]
    p3 = p_ref[...].T[:, :, None]
    eh3 = eh[None, :, :]
    eh_r = p3 * Nb + (1.0 - p3) * eh3
    gate = jnp.tanh(eh3 + eh_r)
    ka = jnp.sum(Nb, axis=2) * jnp.sum(gate, axis=2)
    m = jnp.max(ka, axis=0, keepdims=True)
    e = jnp.exp(ka - m)
    kp = e / jnp.sum(e, axis=0, keepdims=True)
    eNh = jnp.sum(kp[:, :, None] * Nb, axis=0)
    s = _leaky(_dot(eh + eNh, l1w_ref[...]) + l1b_ref[...])
    bi = _leaky(_dot(eh * eNh, l2w_ref[...]) + l2b_ref[...])
    emb = s + bi
    a1 = _leaky(_dot(emb, a1w_ref[...]) + a1b_ref[...])
    g = _dot(a1, a2w_ref[...]) + a2b_ref[...]
    return emb, g


def _kcl_body(nb_ref, eh_ref, p_ref, l1w_ref, l1b_ref, l2w_ref, l2b_ref,
              a1w_ref, a1b_ref, a2w_ref, a2b_ref,
              emb0_ref, g0_ref, ng_ref, nbeta_ref, fcw_ref, fcb_ref,
              lg_ref, yp_ref, yh_ref, emb_s, g_s):
    i = pl.program_id(0)

    @pl.when(i < NBQ)
    def _combine():
        emb, g = _combine_block(nb_ref, eh_ref, p_ref,
                                l1w_ref, l1b_ref, l2w_ref, l2b_ref,
                                a1w_ref, a1b_ref, a2w_ref, a2b_ref)
        emb_s[pl.ds(i * BR, BR), :] = emb
        g_s[pl.ds(i * BR, BR), :] = g

    @pl.when(i == NBQ)
    def _readout():
        m = jnp.maximum(jnp.max(g0_ref[...]), jnp.max(g_s[...]))
        w0 = jnp.exp(g0_ref[...] - m)
        w1 = jnp.exp(g_s[...] - m)
        denom = jnp.sum(w0) + jnp.sum(w1)
        hr = (jnp.sum(w0 * emb0_ref[...], axis=0, keepdims=True)
              + jnp.sum(w1 * emb_s[...], axis=0, keepdims=True)) / denom
        mu = jnp.mean(hr, axis=1, keepdims=True)
        var = jnp.mean((hr - mu) ** 2, axis=1, keepdims=True)
        hn = (hr - mu) / jnp.sqrt(var + 1e-5) * ng_ref[...] + nbeta_ref[...]
        lg = _dot(hn, fcw_ref[...]) + fcb_ref[...]
        lg_ref[...] = lg
        mm = jnp.max(lg, axis=1, keepdims=True)
        ee = jnp.exp(lg - mm)
        yp_ref[...] = ee / jnp.sum(ee, axis=1, keepdims=True)
        yh_ref[...] = jnp.where(lg[:, 1:2] > lg[:, 0:1], 1, 0).astype(jnp.int32)


def kernel(data, CT_data, fc1_W, fc1_b, Wh_W, Wh_b, Wt_W, Wt_b,
           lin1_W, lin1_b, lin2_W, lin2_b, att1_W, att1_b, att2_W, att2_b,
           norm_g, norm_beta, fc_W, fc_b):
    del CT_data
    x0 = jnp.squeeze(data, axis=0)
    r2 = lambda v: v.reshape(1, -1)
    full = lambda a, b: pl.BlockSpec((a, b), lambda i: (0, 0))

    ph1rows = lambda b: pl.BlockSpec(
        (BR, b), lambda i: (jnp.clip(i - NBB, 0, NBB - 1), 0))
    ph2rows = lambda b: pl.BlockSpec(
        (BR, b), lambda i: (jnp.clip(i - 2 * NBB, 0, NBQ - 1), 0))

    e_h, e_t, probs0, idx0 = pl.pallas_call(
        _ka_body,
        grid=(2 * NBB + NBQ,),
        in_specs=[pl.BlockSpec((BR, DIN), lambda i: (jnp.minimum(i, NBB - 1), 0)),
                  full(DIN, DH), full(1, DH),
                  full(DH, DH), full(1, DH),
                  full(DH, DH), full(1, DH)],
        out_specs=[ph1rows(DH), ph1rows(DH), ph2rows(TK), ph2rows(TK)],
        out_shape=[jax.ShapeDtypeStruct((N, DH), jnp.float32),
                   jax.ShapeDtypeStruct((N, DH), jnp.float32),
                   jax.ShapeDtypeStruct((NQ, TK), jnp.float32),
                   jax.ShapeDtypeStruct((NQ, TK), jnp.int32)],
        scratch_shapes=[pltpu.VMEM((N, DH), jnp.float32),
                        pltpu.VMEM((N, DH), jnp.float32),
                        pltpu.VMEM((N, DH), jnp.float32),
                        pltpu.VMEM((1, DH), jnp.float32)],
    )(x0, fc1_W, r2(fc1_b), Wh_W, r2(Wh_b), Wt_W, r2(Wt_b))

    def topk_seg(s):
        return pl.pallas_call(
            _kb_body,
            grid=(NBQ,),
            in_specs=[pl.BlockSpec((BR, DH), lambda i, s=s: (s * NBQ + i, 0)),
                      full(N, DH)],
            out_specs=[pl.BlockSpec((BR, TK), lambda i: (i, 0)),
                       pl.BlockSpec((BR, TK), lambda i: (i, 0))],
            out_shape=[jax.ShapeDtypeStruct((NQ, TK), jnp.float32),
                       jax.ShapeDtypeStruct((NQ, TK), jnp.int32)],
        )(e_h, e_t)

    probs = [probs0] + [None] * (NSEG - 1)
    idxs = [idx0] + [None] * (NSEG - 1)
    for s in range(1, NSEG):
        probs[s], idxs[s] = topk_seg(s)

    nbs = [_gather_rows(e_t, idxs[s].T.reshape(_BQ)).reshape(TK, NQ, DH)
           for s in range(NSEG)]

    wspecs = [full(DH, DH), full(1, DH), full(DH, DH), full(1, DH),
              full(DH, DH // 2), full(1, DH // 2), full(DH // 2, 1),
              full(1, 1)]
    wargs = (lin1_W, r2(lin1_b), lin2_W, r2(lin2_b),
             att1_W, r2(att1_b), att2_W, r2(att2_b))

    def combine_seg(s):
        return pl.pallas_call(
            _kc_body,
            grid=(NBQ,),
            in_specs=[pl.BlockSpec((TK, BR, DH), lambda i: (0, i, 0)),
                      pl.BlockSpec((BR, DH), lambda i, s=s: (s * NBQ + i, 0)),
                      pl.BlockSpec((BR, TK), lambda i: (i, 0))] + wspecs,
            out_specs=[pl.BlockSpec((BR, DH), lambda i: (i, 0)),
                       pl.BlockSpec((BR, 1), lambda i: (i, 0))],
            out_shape=[jax.ShapeDtypeStruct((NQ, DH), jnp.float32),
                       jax.ShapeDtypeStruct((NQ, 1), jnp.float32)],
        )(nbs[s], e_h, probs[s], *wargs)

    emb0, g0 = combine_seg(0)

    logits, y_prob, y_hat = pl.pallas_call(
        _kcl_body,
        grid=(NBQ + 1,),
        in_specs=[pl.BlockSpec((TK, BR, DH),
                               lambda i: (0, jnp.minimum(i, NBQ - 1), 0)),
                  pl.BlockSpec((BR, DH),
                               lambda i: (NBQ + jnp.minimum(i, NBQ - 1), 0)),
                  pl.BlockSpec((BR, TK), lambda i: (jnp.minimum(i, NBQ - 1), 0))]
                 + wspecs
                 + [full(NQ, DH), full(NQ, 1),
                    full(1, DH), full(1, DH), full(DH, 2), full(1, 2)],
        out_specs=[full(1, 2), full(1, 2), full(1, 1)],
        out_shape=[jax.ShapeDtypeStruct((1, 2), jnp.float32),
                   jax.ShapeDtypeStruct((1, 2), jnp.float32),
                   jax.ShapeDtypeStruct((1, 1), jnp.int32)],
        scratch_shapes=[pltpu.VMEM((NQ, DH), jnp.float32),
                        pltpu.VMEM((NQ, 1), jnp.float32)],
    )(nbs[1], e_h, probs[1], *wargs, emb0, g0,
      r2(norm_g), r2(norm_beta), fc_W, r2(fc_b))

    return (logits, y_prob, y_hat)


# speedup vs baseline: 1.1347x; 1.0329x over previous
import functools

import jax
import jax.numpy as jnp
from jax import lax
from jax.experimental import pallas as pl
from jax.experimental.pallas import tpu as pltpu
from jax.experimental.pallas import tpu_sc as plsc

N = 4096
DIN = 384
DH = 512
TK = 6
BR = 512
NBB = N // BR
NSEG = 2
NQ = N // NSEG
NBQ = NQ // BR

_NC = 2
_NS = 16
_NW = _NC * _NS
_BQ = NQ * TK
_BPW = _BQ // _NW
_CH = 64
_NCHUNK = _BPW // _CH
_KBUF = 3


def _leaky(x):
    return jnp.where(x >= 0, x, 0.01 * x)


def _dot(a, b):
    return jnp.dot(a, b, preferred_element_type=jnp.float32)


def _topk_block(eh, et_full):
    scale = DH ** (-0.5)
    logits = lax.dot_general(eh * scale, et_full,
                             (((1,), (1,)), ((), ())),
                             preferred_element_type=jnp.float32)
    iota = lax.broadcasted_iota(jnp.int32, logits.shape, 1)
    vals, idxs = [], []
    for _ in range(TK):
        m = jnp.max(logits, axis=1, keepdims=True)
        jj = jnp.min(jnp.where(logits >= m, iota, N), axis=1, keepdims=True)
        vals.append(m)
        idxs.append(jj)
        logits = jnp.where(iota == jj, -jnp.inf, logits)
    v = jnp.concatenate(vals, axis=1)
    ji = jnp.concatenate(idxs, axis=1)
    e = jnp.exp(v - v[:, 0:1])
    return e / jnp.sum(e, axis=1, keepdims=True), ji


def _ka_body(data_ref, fc1w_ref, fc1b_ref, whw_ref, whb_ref, wtw_ref, wtb_ref,
             eh_ref, et_ref, prob_ref, idx_ref,
             h1_s, eh_s, et_s, csum_s):
    i = pl.program_id(0)

    @pl.when(i < NBB)
    def _phase0():
        h = _leaky(_dot(data_ref[...], fc1w_ref[...]) + fc1b_ref[...])
        h1_s[pl.ds(i * BR, BR), :] = h

        @pl.when(i == 0)
        def _():
            csum_s[...] = jnp.zeros_like(csum_s)

        csum_s[...] += jnp.sum(h, axis=0, keepdims=True)

    @pl.when(jnp.logical_and(i >= NBB, i < 2 * NBB))
    def _phase1():
        j = i - NBB
        x = (h1_s[pl.ds(j * BR, BR), :] + csum_s[...] * (1.0 / N)) * 0.5
        eh = _dot(x, whw_ref[...]) + whb_ref[...]
        et = _dot(x, wtw_ref[...]) + wtb_ref[...]
        eh_s[pl.ds(j * BR, BR), :] = eh
        et_s[pl.ds(j * BR, BR), :] = et
        eh_ref[...] = eh
        et_ref[...] = et

    @pl.when(i >= 2 * NBB)
    def _phase2():
        j = i - 2 * NBB
        p, ji = _topk_block(eh_s[pl.ds(j * BR, BR), :], et_s[...])
        prob_ref[...] = p
        idx_ref[...] = ji


def _kb_body(eh_ref, et_ref, prob_ref, idx_ref):
    p, ji = _topk_block(eh_ref[...], et_ref[...])
    prob_ref[...] = p
    idx_ref[...] = ji


def _sc_gather_body(table_hbm, idx_hbm, out_hbm, idx_v, *bufs_sems):
    bufs = bufs_sems[:_KBUF]
    gsem = bufs_sems[_KBUF:2 * _KBUF]
    ssem = bufs_sems[2 * _KBUF:3 * _KBUF]
    wid = lax.axis_index("s") * _NC + lax.axis_index("c")
    base = wid * _BPW
    pltpu.sync_copy(idx_hbm.at[pl.ds(base, _BPW)], idx_v)
    gh = [None] * _KBUF
    sh = [None] * _KBUF
    for b in range(_KBUF):
        gh[b] = pltpu.async_copy(
            table_hbm.at[idx_v.at[pl.ds(b * _CH, _CH)]], bufs[b], gsem[b])
    for c in range(_NCHUNK):
        b = c % _KBUF
        gh[b].wait()
        sh[b] = pltpu.async_copy(
            bufs[b], out_hbm.at[pl.ds(base + c * _CH, _CH)], ssem[b])
        if c + _KBUF < _NCHUNK:
            sh[b].wait()
            gh[b] = pltpu.async_copy(
                table_hbm.at[idx_v.at[pl.ds((c + _KBUF) * _CH, _CH)]],
                bufs[b], gsem[b])
    for c in range(max(0, _NCHUNK - _KBUF), _NCHUNK):
        sh[c % _KBUF].wait()


def _gather_rows(table, idx):
    mesh = plsc.VectorSubcoreMesh(
        core_axis_name="c", subcore_axis_name="s",
        num_cores=_NC, num_subcores=_NS)
    f = functools.partial(
        pl.kernel, mesh=mesh,
        out_type=jax.ShapeDtypeStruct((_BQ, DH), jnp.float32),
        scratch_types=[pltpu.VMEM((_BPW,), jnp.int32)]
        + [pltpu.VMEM((_CH, DH), jnp.float32) for _ in range(_KBUF)]
        + [pltpu.SemaphoreType.DMA for _ in range(2 * _KBUF)],
    )(_sc_gather_body)
    return f(table, idx)


def _kc_body(nb_ref, eh_ref, p_ref, l1w_ref, l1b_ref, l2w_ref, l2b_ref,
             a1w_ref, a1b_ref, a2w_ref, a2b_ref, emb_ref, g_ref):
    emb, g = _combine_block(nb_ref, eh_ref, p_ref,
                            l1w_ref, l1b_ref, l2w_ref, l2b_ref,
                            a1w_ref, a1b_ref, a2w_ref, a2b_ref)
    emb_ref[...] = emb
    g_ref[...] = g


def _combine_block(nb_ref, eh_ref, p_ref, l1w_ref, l1b_ref, l2w_ref, l2b_ref,
                   a1w_ref, a1b_ref, a2w_ref, a2b_ref):
    Nb = nb_ref[...]
    eh = eh_ref[...]
    p3 = p_ref[...].T[:, :, None]
    eh3 = eh[None, :, :]
    eh_r = p3 * Nb + (1.0 - p3) * eh3
    gate = jnp.tanh(eh3 + eh_r)
    ka = jnp.sum(Nb, axis=2) * jnp.sum(gate, axis=2)
    m = jnp.max(ka, axis=0, keepdims=True)
    e = jnp.exp(ka - m)
    kp = e / jnp.sum(e, axis=0, keepdims=True)
    eNh = jnp.sum(kp[:, :, None] * Nb, axis=0)
    s = _leaky(_dot(eh + eNh, l1w_ref[...]) + l1b_ref[...])
    bi = _leaky(_dot(eh * eNh, l2w_ref[...]) + l2b_ref[...])
    emb = s + bi
    a1 = _leaky(_dot(emb, a1w_ref[...]) + a1b_ref[...])
    g = _dot(a1, a2w_ref[...]) + a2b_ref[...]
    return emb, g


def _kcl_body(nb_ref, eh_ref, p_ref, l1w_ref, l1b_ref, l2w_ref, l2b_ref,
              a1w_ref, a1b_ref, a2w_ref, a2b_ref,
              emb0_ref, g0_ref, ng_ref, nbeta_ref, fcw_ref, fcb_ref,
              lg_ref, yp_ref, yh_ref, emb_s, g_s):
    i = pl.program_id(0)

    @pl.when(i < NBQ)
    def _combine():
        emb, g = _combine_block(nb_ref, eh_ref, p_ref,
                                l1w_ref, l1b_ref, l2w_ref, l2b_ref,
                                a1w_ref, a1b_ref, a2w_ref, a2b_ref)
        emb_s[pl.ds(i * BR, BR), :] = emb
        g_s[pl.ds(i * BR, BR), :] = g

    @pl.when(i == NBQ)
    def _readout():
        m = jnp.maximum(jnp.max(g0_ref[...]), jnp.max(g_s[...]))
        w0 = jnp.exp(g0_ref[...] - m)
        w1 = jnp.exp(g_s[...] - m)
        denom = jnp.sum(w0) + jnp.sum(w1)
        hr = (jnp.sum(w0 * emb0_ref[...], axis=0, keepdims=True)
              + jnp.sum(w1 * emb_s[...], axis=0, keepdims=True)) / denom
        mu = jnp.mean(hr, axis=1, keepdims=True)
        var = jnp.mean((hr - mu) ** 2, axis=1, keepdims=True)
        hn = (hr - mu) / jnp.sqrt(var + 1e-5) * ng_ref[...] + nbeta_ref[...]
        lg = _dot(hn, fcw_ref[...]) + fcb_ref[...]
        lg_ref[...] = lg
        mm = jnp.max(lg, axis=1, keepdims=True)
        ee = jnp.exp(lg - mm)
        yp_ref[...] = ee / jnp.sum(ee, axis=1, keepdims=True)
        yh_ref[...] = jnp.where(lg[:, 1:2] > lg[:, 0:1], 1, 0).astype(jnp.int32)


def kernel(data, CT_data, fc1_W, fc1_b, Wh_W, Wh_b, Wt_W, Wt_b,
           lin1_W, lin1_b, lin2_W, lin2_b, att1_W, att1_b, att2_W, att2_b,
           norm_g, norm_beta, fc_W, fc_b):
    del CT_data
    x0 = jnp.squeeze(data, axis=0)
    r2 = lambda v: v.reshape(1, -1)
    full = lambda a, b: pl.BlockSpec((a, b), lambda i: (0, 0))

    ph1rows = lambda b: pl.BlockSpec(
        (BR, b), lambda i: (jnp.clip(i - NBB, 0, NBB - 1), 0))
    ph2rows = lambda b: pl.BlockSpec(
        (BR, b), lambda i: (jnp.clip(i - 2 * NBB, 0, NBQ - 1), 0))

    e_h, e_t, probs0, idx0 = pl.pallas_call(
        _ka_body,
        grid=(2 * NBB + NBQ,),
        in_specs=[pl.BlockSpec((BR, DIN), lambda i: (jnp.minimum(i, NBB - 1), 0)),
                  full(DIN, DH), full(1, DH),
                  full(DH, DH), full(1, DH),
                  full(DH, DH), full(1, DH)],
        out_specs=[ph1rows(DH), ph1rows(DH), ph2rows(TK), ph2rows(TK)],
        out_shape=[jax.ShapeDtypeStruct((N, DH), jnp.float32),
                   jax.ShapeDtypeStruct((N, DH), jnp.float32),
                   jax.ShapeDtypeStruct((NQ, TK), jnp.float32),
                   jax.ShapeDtypeStruct((NQ, TK), jnp.int32)],
        scratch_shapes=[pltpu.VMEM((N, DH), jnp.float32),
                        pltpu.VMEM((N, DH), jnp.float32),
                        pltpu.VMEM((N, DH), jnp.float32),
                        pltpu.VMEM((1, DH), jnp.float32)],
    )(x0, fc1_W, r2(fc1_b), Wh_W, r2(Wh_b), Wt_W, r2(Wt_b))

    def topk_seg(s):
        return pl.pallas_call(
            _kb_body,
            grid=(NBQ,),
            in_specs=[pl.BlockSpec((BR, DH), lambda i, s=s: (s * NBQ + i, 0)),
                      full(N, DH)],
            out_specs=[pl.BlockSpec((BR, TK), lambda i: (i, 0)),
                       pl.BlockSpec((BR, TK), lambda i: (i, 0))],
            out_shape=[jax.ShapeDtypeStruct((NQ, TK), jnp.float32),
                       jax.ShapeDtypeStruct((NQ, TK), jnp.int32)],
        )(e_h, e_t)

    probs = [probs0] + [None] * (NSEG - 1)
    idxs = [idx0] + [None] * (NSEG - 1)
    for s in range(1, NSEG):
        probs[s], idxs[s] = topk_seg(s)

    nbs = [_gather_rows(e_t, idxs[s].T.reshape(_BQ)).reshape(TK, NQ, DH)
           for s in range(NSEG)]

    wspecs = [full(DH, DH), full(1, DH), full(DH, DH), full(1, DH),
              full(DH, DH // 2), full(1, DH // 2), full(DH // 2, 1),
              full(1, 1)]
    wargs = (lin1_W, r2(lin1_b), lin2_W, r2(lin2_b),
             att1_W, r2(att1_b), att2_W, r2(att2_b))

    def combine_seg(s):
        return pl.pallas_call(
            _kc_body,
            grid=(NBQ,),
            in_specs=[pl.BlockSpec((TK, BR, DH), lambda i: (0, i, 0)),
                      pl.BlockSpec((BR, DH), lambda i, s=s: (s * NBQ + i, 0)),
                      pl.BlockSpec((BR, TK), lambda i: (i, 0))] + wspecs,
            out_specs=[pl.BlockSpec((BR, DH), lambda i: (i, 0)),
                       pl.BlockSpec((BR, 1), lambda i: (i, 0))],
            out_shape=[jax.ShapeDtypeStruct((NQ, DH), jnp.float32),
                       jax.ShapeDtypeStruct((NQ, 1), jnp.float32)],
        )(nbs[s], e_h, probs[s], *wargs)

    emb0, g0 = combine_seg(0)

    logits, y_prob, y_hat = pl.pallas_call(
        _kcl_body,
        grid=(NBQ + 1,),
        in_specs=[pl.BlockSpec((TK, BR, DH),
                               lambda i: (0, jnp.minimum(i, NBQ - 1), 0)),
                  pl.BlockSpec((BR, DH),
                               lambda i: (NBQ + jnp.minimum(i, NBQ - 1), 0)),
                  pl.BlockSpec((BR, TK), lambda i: (jnp.minimum(i, NBQ - 1), 0))]
                 + wspecs
                 + [full(NQ, DH), full(NQ, 1),
                    full(1, DH), full(1, DH), full(DH, 2), full(1, 2)],
        out_specs=[full(1, 2), full(1, 2), full(1, 1)],
        out_shape=[jax.ShapeDtypeStruct((1, 2), jnp.float32),
                   jax.ShapeDtypeStruct((1, 2), jnp.float32),
                   jax.ShapeDtypeStruct((1, 1), jnp.int32)],
        scratch_shapes=[pltpu.VMEM((NQ, DH), jnp.float32),
                        pltpu.VMEM((NQ, 1), jnp.float32)],
    )(nbs[1], e_h, probs[1], *wargs, emb0, g0,
      r2(norm_g), r2(norm_beta), fc_W, r2(fc_b))

    return (logits, y_prob, y_hat)
